# baseline (speedup 1.0000x reference)
import functools

import jax
import jax.numpy as jnp
from jax import lax
from jax.experimental import pallas as pl
from jax.experimental.pallas import tpu as pltpu
from jax.experimental.pallas import tpu_sc as plsc

B = 16384
V = 100000
D = 16
N_CAT = 4

NC = 2
NS = 16
L = 16
NW = NC * NS
BPW = B // NW
CH = 128
NCH = BPW // CH
VB = V // 8
RPB = 128 // D

_mesh = plsc.VectorSubcoreMesh(core_axis_name="c", subcore_axis_name="s")


@functools.partial(
    pl.kernel,
    out_type=jax.ShapeDtypeStruct((B * D,), jnp.float32),
    mesh=_mesh,
    compiler_params=pltpu.CompilerParams(
        needs_layout_passes=False, use_tc_tiling_on_sc=True),
    scratch_types=[
        pltpu.VMEM((NCH, CH), jnp.int32),
        pltpu.VMEM((NCH, CH), jnp.int32),
        pltpu.VMEM((NCH, CH, 128), jnp.float32),
        pltpu.VMEM((BPW * D,), jnp.float32),
        pltpu.SemaphoreType.DMA,
    ],
)
def _gather_sc(table, cat, out, idx_v, bidx_v, bufs, rows_v, sem):
    wid = lax.axis_index("s") * NC + lax.axis_index("c")
    pltpu.sync_copy(cat.at[wid], idx_v)
    for c in range(NCH):
        for g in range(CH // L):
            v = idx_v[c, pl.ds(g * L, L)]
            bidx_v[c, pl.ds(g * L, L)] = lax.shift_right_logical(v, 3)
    copies = [
        pltpu.async_copy(table.at[bidx_v.at[c]], bufs.at[c], sem)
        for c in range(NCH)
    ]
    lane = lax.iota(jnp.int32, L)
    seven = lane * 0 + 7
    for c in range(NCH):
        copies[c].wait()
        for g in range(CH // L):
            idxs = idx_v[c, pl.ds(g * L, L)]
            off = lax.mul(lax.bitwise_and(idxs, seven), seven * 0 + D)
            rows16 = lane + g * L
            dstbase = lax.mul(rows16 + c * CH, seven * 0 + D)
            for j in range(D):
                vals = plsc.load_gather(bufs.at[c], [rows16, off + j])
                plsc.store_scatter(rows_v, [dstbase + j], vals)
    pltpu.sync_copy(rows_v, out.at[pl.ds(wid * (BPW * D), BPW * D)])


@functools.partial(
    pl.kernel,
    out_type=jax.ShapeDtypeStruct((B * 2,), jnp.float32),
    mesh=_mesh,
    compiler_params=pltpu.CompilerParams(
        needs_layout_passes=False, use_tc_tiling_on_sc=True),
    scratch_types=[
        pltpu.VMEM((BPW,), jnp.float32),
        pltpu.VMEM((BPW,), jnp.float32),
        pltpu.VMEM((BPW * 2,), jnp.float32),
    ],
)
def _concat_sc(n0_hbm, n1_hbm, out_num, n0_v, n1_v, nbuf):
    wid = lax.axis_index("s") * NC + lax.axis_index("c")
    base = wid * BPW
    pltpu.sync_copy(n0_hbm.at[wid], n0_v)
    pltpu.sync_copy(n1_hbm.at[wid], n1_v)
    lane = lax.iota(jnp.int32, L)
    for i in range(BPW // L):
        flat = (lane + i * L) * 2
        v0 = n0_v[pl.ds(i * L, L)]
        v1 = n1_v[pl.ds(i * L, L)]
        plsc.store_scatter(nbuf, [flat], v0)
        plsc.store_scatter(nbuf, [flat + 1], v1)
    pltpu.sync_copy(nbuf, out_num.at[pl.ds(base * 2, BPW * 2)])


def kernel(num_0, num_1, emb_cat_0, emb_cat_1, emb_cat_2, emb_cat_3,
           cat_0, cat_1, cat_2, cat_3):
    n0 = num_0.astype(jnp.float32).reshape(NW, BPW)
    n1 = num_1.astype(jnp.float32).reshape(NW, BPW)
    out_num = _concat_sc(n0, n1)
    es = []
    for tbl, cat in ((emb_cat_0, cat_0), (emb_cat_1, cat_1),
                     (emb_cat_2, cat_2), (emb_cat_3, cat_3)):
        flat = _gather_sc(tbl.reshape(VB, 128), cat.reshape(NW, NCH, CH))
        es.append(flat.reshape(B, D))
    return (out_num.reshape(B, 2), es[0], es[1], es[2], es[3])


# speedup vs baseline: 1.0222x; 1.0222x over previous
import functools

import jax
import jax.numpy as jnp
from jax import lax
from jax.experimental import pallas as pl
from jax.experimental.pallas import tpu as pltpu
from jax.experimental.pallas import tpu_sc as plsc

B = 16384
V = 100000
D = 16
N_CAT = 4

NC = 2
NS = 16
L = 16
NW = NC * NS
BPW = B // NW
CH = 128
NCH = BPW // CH

_mesh = plsc.VectorSubcoreMesh(core_axis_name="c", subcore_axis_name="s")
_params = pltpu.CompilerParams(
    needs_layout_passes=False, use_tc_tiling_on_sc=False)


@functools.partial(
    pl.kernel,
    out_type=(jax.ShapeDtypeStruct((B, D), jnp.float32),
              jax.ShapeDtypeStruct((B * 2,), jnp.float32)),
    mesh=_mesh,
    compiler_params=_params,
    scratch_types=[
        pltpu.VMEM((NCH, CH), jnp.int32),
        pltpu.VMEM((BPW, D), jnp.float32),
        pltpu.VMEM((BPW,), jnp.float32),
        pltpu.VMEM((BPW,), jnp.float32),
        pltpu.VMEM((BPW * 2,), jnp.float32),
        pltpu.SemaphoreType.DMA,
    ],
)
def _gather_num_sc(table, cat, n0_hbm, n1_hbm, out, out_num,
                   idx_v, rows_v, n0_v, n1_v, nbuf, sem):
    wid = lax.axis_index("s") * NC + lax.axis_index("c")
    base = wid * BPW
    pltpu.sync_copy(cat.at[wid], idx_v)
    pltpu.sync_copy(n0_hbm.at[wid], n0_v)
    pltpu.sync_copy(n1_hbm.at[wid], n1_v)
    copies = []
    for j in range(NCH):
        copies.append(
            pltpu.async_copy(
                table.at[idx_v.at[j]],
                rows_v.at[pl.ds(j * CH, CH)],
                sem,
            )
        )
    lane = lax.iota(jnp.int32, L)
    for i in range(BPW // L):
        flat = (lane + i * L) * 2
        v0 = n0_v[pl.ds(i * L, L)]
        v1 = n1_v[pl.ds(i * L, L)]
        plsc.store_scatter(nbuf, [flat], v0)
        plsc.store_scatter(nbuf, [flat + 1], v1)
    pltpu.sync_copy(nbuf, out_num.at[pl.ds(base * 2, BPW * 2)])
    for c in copies:
        c.wait()
    pltpu.sync_copy(rows_v, out.at[pl.ds(base, BPW)])


@functools.partial(
    pl.kernel,
    out_type=jax.ShapeDtypeStruct((B, D), jnp.float32),
    mesh=_mesh,
    compiler_params=_params,
    scratch_types=[
        pltpu.VMEM((NCH, CH), jnp.int32),
        pltpu.VMEM((BPW, D), jnp.float32),
        pltpu.SemaphoreType.DMA,
    ],
)
def _gather_sc(table, cat, out, idx_v, rows_v, sem):
    wid = lax.axis_index("s") * NC + lax.axis_index("c")
    base = wid * BPW
    pltpu.sync_copy(cat.at[wid], idx_v)
    copies = []
    for j in range(NCH):
        copies.append(
            pltpu.async_copy(
                table.at[idx_v.at[j]],
                rows_v.at[pl.ds(j * CH, CH)],
                sem,
            )
        )
    for c in copies:
        c.wait()
    pltpu.sync_copy(rows_v, out.at[pl.ds(base, BPW)])


def kernel(num_0, num_1, emb_cat_0, emb_cat_1, emb_cat_2, emb_cat_3,
           cat_0, cat_1, cat_2, cat_3):
    n0 = num_0.astype(jnp.float32).reshape(NW, BPW)
    n1 = num_1.astype(jnp.float32).reshape(NW, BPW)
    e0, out_num = _gather_num_sc(
        emb_cat_0, cat_0.reshape(NW, NCH, CH), n0, n1)
    es = [e0]
    for tbl, cat in ((emb_cat_1, cat_1), (emb_cat_2, cat_2),
                     (emb_cat_3, cat_3)):
        es.append(_gather_sc(tbl, cat.reshape(NW, NCH, CH)))
    return (out_num.reshape(B, 2), es[0], es[1], es[2], es[3])


# speedup vs baseline: 1.0266x; 1.0043x over previous
import functools

import jax
import jax.numpy as jnp
from jax import lax
from jax.experimental import pallas as pl
from jax.experimental.pallas import tpu as pltpu
from jax.experimental.pallas import tpu_sc as plsc

B = 16384
V = 100000
D = 16
N_CAT = 4

NC = 2
NS = 16
L = 16
NW = NC * NS
BPW = B // NW
CH = 128
NCH = BPW // CH

_mesh = plsc.VectorSubcoreMesh(core_axis_name="c", subcore_axis_name="s")
_params = pltpu.CompilerParams(
    needs_layout_passes=False, use_tc_tiling_on_sc=False)


@functools.partial(
    pl.kernel,
    out_type=jax.ShapeDtypeStruct((B, D), jnp.float32),
    mesh=_mesh,
    compiler_params=_params,
    scratch_types=[
        pltpu.VMEM((NCH, CH), jnp.int32),
        pltpu.VMEM((BPW, D), jnp.float32),
        pltpu.SemaphoreType.DMA,
    ],
)
def _gather_sc(table, cat, out, idx_v, rows_v, sem):
    wid = lax.axis_index("s") * NC + lax.axis_index("c")
    base = wid * BPW
    pltpu.sync_copy(cat.at[wid], idx_v)
    copies = []
    for j in range(NCH):
        copies.append(
            pltpu.async_copy(
                table.at[idx_v.at[j]],
                rows_v.at[pl.ds(j * CH, CH)],
                sem,
            )
        )
    for c in copies:
        c.wait()
    pltpu.sync_copy(rows_v, out.at[pl.ds(base, BPW)])


@functools.partial(
    pl.kernel,
    out_type=jax.ShapeDtypeStruct((B * 2,), jnp.float32),
    mesh=_mesh,
    compiler_params=_params,
    scratch_types=[
        pltpu.VMEM((BPW,), jnp.float32),
        pltpu.VMEM((BPW,), jnp.float32),
        pltpu.VMEM((BPW * 2,), jnp.float32),
    ],
)
def _concat_sc(n0_hbm, n1_hbm, out_num, n0_v, n1_v, nbuf):
    wid = lax.axis_index("s") * NC + lax.axis_index("c")
    base = wid * BPW
    pltpu.sync_copy(n0_hbm.at[wid], n0_v)
    pltpu.sync_copy(n1_hbm.at[wid], n1_v)
    lane = lax.iota(jnp.int32, L)
    for i in range(BPW // L):
        flat = (lane + i * L) * 2
        v0 = n0_v[pl.ds(i * L, L)]
        v1 = n1_v[pl.ds(i * L, L)]
        plsc.store_scatter(nbuf, [flat], v0)
        plsc.store_scatter(nbuf, [flat + 1], v1)
    pltpu.sync_copy(nbuf, out_num.at[pl.ds(base * 2, BPW * 2)])


def kernel(num_0, num_1, emb_cat_0, emb_cat_1, emb_cat_2, emb_cat_3,
           cat_0, cat_1, cat_2, cat_3):
    n0 = num_0.astype(jnp.float32).reshape(NW, BPW)
    n1 = num_1.astype(jnp.float32).reshape(NW, BPW)
    out_num = _concat_sc(n0, n1)
    es = []
    for tbl, cat in ((emb_cat_0, cat_0), (emb_cat_1, cat_1),
                     (emb_cat_2, cat_2), (emb_cat_3, cat_3)):
        es.append(_gather_sc(tbl, cat.reshape(NW, NCH, CH)))
    return (out_num.reshape(B, 2), es[0], es[1], es[2], es[3])
